# baseline (speedup 1.0000x reference)
import functools

import jax
import jax.numpy as jnp
from jax import lax
from jax.experimental import pallas as pl
from jax.experimental.pallas import tpu as pltpu
from jax.experimental.pallas import tpu_sc as plsc

VOCAB = 1000000
EMBED = 64
MAXLEN = 200
BATCH = 4096
NW = 32
BB = BATCH // NW
NUNITS = MAXLEN
NBUF = 4
NLANES = 16

_mesh = plsc.VectorSubcoreMesh(core_axis_name="c", subcore_axis_name="s")


@functools.partial(
    pl.kernel,
    out_type=jax.ShapeDtypeStruct((MAXLEN, EMBED, BATCH), jnp.float32),
    mesh=_mesh,
    scratch_types=[
        pltpu.VMEM((MAXLEN, EMBED), jnp.float32),
        [pltpu.VMEM((BB,), jnp.int32) for _ in range(NBUF)],
        [pltpu.VMEM((BB, EMBED), jnp.float32) for _ in range(NBUF)],
        [pltpu.VMEM((EMBED, BB), jnp.float32) for _ in range(NBUF)],
        [pltpu.SemaphoreType.DMA for _ in range(NBUF)],
        [pltpu.SemaphoreType.DMA for _ in range(NBUF)],
        [pltpu.SemaphoreType.DMA for _ in range(NBUF)],
    ],
    compiler_params=pltpu.CompilerParams(use_tc_tiling_on_sc=False,
                                         needs_layout_passes=False),
)
def _embed_sc(xt_hbm, table_hbm, fix_hbm, out_hbm,
              fix_v, idx_v, rows_v, outb_v, isem, gsem, ssem):
    wid = lax.axis_index("s") * 2 + lax.axis_index("c")
    b0 = wid * BB
    pltpu.sync_copy(fix_hbm, fix_v)

    def idx_start(b, t):
        pltpu.async_copy(xt_hbm.at[t, pl.ds(b0, BB)], idx_v[b], isem[b])

    def idx_wait(b, t):
        pltpu.make_async_copy(xt_hbm.at[t, pl.ds(b0, BB)], idx_v[b],
                              isem[b]).wait()

    def gather_start(b):
        pltpu.async_copy(table_hbm.at[idx_v[b]], rows_v[b], gsem[b])

    def gather_wait(b):
        pltpu.make_async_copy(table_hbm.at[idx_v[b]], rows_v[b],
                              gsem[b]).wait()

    def store_start(b, t):
        pltpu.async_copy(outb_v[b], out_hbm.at[t, :, pl.ds(b0, BB)], ssem[b])

    def store_wait(b, t):
        pltpu.make_async_copy(outb_v[b], out_hbm.at[t, :, pl.ds(b0, BB)],
                              ssem[b]).wait()

    def add_transpose(b, t):
        rows = rows_v[b]
        ob = outb_v[b]
        lanes = lax.iota(jnp.int32, NLANES)
        tsplat = jnp.full((NLANES,), t, jnp.int32)

        def e_body(e, _):
            esplat = jnp.full((NLANES,), e, jnp.int32)
            fsp = plsc.load_gather(fix_v, [tsplat, esplat])
            for bg in range(BB // NLANES):
                bid = lanes + (bg * NLANES)
                vals = plsc.load_gather(rows, [bid, esplat])
                ob[e, pl.ds(bg * NLANES, NLANES)] = vals + fsp
            return ()

        lax.fori_loop(0, EMBED, e_body, ())

    for b in range(3):
        idx_start(b, b)
    for b in range(2):
        idx_wait(b, b)
        gather_start(b)

    def group_body(g, _):
        for b in range(NBUF):
            t = g * NBUF + b
            bi = (b + 3) % NBUF

            @pl.when(t + 3 < NUNITS)
            def _():
                idx_start(bi, t + 3)

            gather_wait(b)
            add_transpose(b, t)
            store_start(b, t)

            bg = (b + 2) % NBUF

            @pl.when(t + 2 < NUNITS)
            def _():
                @pl.when(t >= 2)
                def _():
                    store_wait(bg, t - 2)
                idx_wait(bg, t + 2)
                gather_start(bg)
        return ()

    lax.fori_loop(0, NUNITS // NBUF, group_body, ())

    for b in range(NBUF):
        store_wait(b, NUNITS - NBUF + b)


def kernel(x, input_table, fix_embedding):
    out_teb = _embed_sc(x.T, input_table, fix_embedding)
    return jnp.transpose(out_teb, (2, 0, 1))


# speedup vs baseline: 1.0930x; 1.0930x over previous
import functools

import jax
import jax.numpy as jnp
from jax import lax
from jax.experimental import pallas as pl
from jax.experimental.pallas import tpu as pltpu
from jax.experimental.pallas import tpu_sc as plsc

VOCAB = 1000000
EMBED = 64
MAXLEN = 200
BATCH = 4096
NW = 32
BB = BATCH // NW
NBUF = 4
NLANES = 16
EBLK = EMBED // 8

_mesh = plsc.VectorSubcoreMesh(core_axis_name="c", subcore_axis_name="s")


@functools.partial(
    pl.kernel,
    out_type=jax.ShapeDtypeStruct((MAXLEN * EBLK, NW, 8, BB), jnp.float32),
    mesh=_mesh,
    scratch_types=[
        pltpu.VMEM((MAXLEN, EMBED), jnp.float32),
        [pltpu.VMEM((BB,), jnp.int32) for _ in range(NBUF)],
        [pltpu.VMEM((BB, EMBED), jnp.float32) for _ in range(NBUF)],
        [pltpu.VMEM((EBLK, 1, 8, BB), jnp.float32) for _ in range(NBUF)],
        [pltpu.SemaphoreType.DMA for _ in range(NBUF)],
        [pltpu.SemaphoreType.DMA for _ in range(NBUF)],
        [pltpu.SemaphoreType.DMA for _ in range(NBUF)],
    ],
    compiler_params=pltpu.CompilerParams(use_tc_tiling_on_sc=False,
                                         needs_layout_passes=False),
)
def _embed_sc(xt_hbm, table_hbm, fix_hbm, out_hbm,
              fix_v, idx_v, rows_v, outb_v, isem, gsem, ssem):
    wid = lax.axis_index("s") * 2 + lax.axis_index("c")
    b0 = wid * BB
    pltpu.sync_copy(fix_hbm, fix_v)

    def idx_start(b, t):
        pltpu.async_copy(xt_hbm.at[t, pl.ds(b0, BB)], idx_v[b], isem[b])

    def idx_wait(b, t):
        pltpu.make_async_copy(xt_hbm.at[t, pl.ds(b0, BB)], idx_v[b],
                              isem[b]).wait()

    def gather_start(b):
        pltpu.async_copy(table_hbm.at[idx_v[b]], rows_v[b], gsem[b])

    def gather_wait(b):
        pltpu.make_async_copy(table_hbm.at[idx_v[b]], rows_v[b],
                              gsem[b]).wait()

    def out_slice(t):
        return out_hbm.at[pl.ds(t * EBLK, EBLK), pl.ds(wid, 1)]

    def store_start(b, t):
        pltpu.async_copy(outb_v[b], out_slice(t), ssem[b])

    def store_wait(b, t):
        pltpu.make_async_copy(outb_v[b], out_slice(t), ssem[b]).wait()

    def add_transpose(b, t):
        rows = rows_v[b]
        ob = outb_v[b]
        lanes = lax.iota(jnp.int32, NLANES)
        tsplat = jnp.full((NLANES,), t, jnp.int32)

        def eb_body(eb, _):
            for ei in range(8):
                e = eb * 8 + ei
                esplat = jnp.full((NLANES,), e, jnp.int32)
                fsp = plsc.load_gather(fix_v, [tsplat, esplat])
                for bg in range(BB // NLANES):
                    bid = lanes + (bg * NLANES)
                    vals = plsc.load_gather(rows, [bid, esplat])
                    ob[eb, 0, ei, pl.ds(bg * NLANES, NLANES)] = vals + fsp
            return ()

        lax.fori_loop(0, EBLK, eb_body, ())

    for b in range(3):
        idx_start(b, b)
    for b in range(2):
        idx_wait(b, b)
        gather_start(b)

    def group_body(g, _):
        for b in range(NBUF):
            t = g * NBUF + b
            bi = (b + 3) % NBUF

            @pl.when(t + 3 < MAXLEN)
            def _():
                idx_start(bi, t + 3)

            gather_wait(b)
            add_transpose(b, t)
            store_start(b, t)

            bg = (b + 2) % NBUF

            @pl.when(t + 2 < MAXLEN)
            def _():
                @pl.when(t >= 2)
                def _():
                    store_wait(bg, t - 2)
                idx_wait(bg, t + 2)
                gather_start(bg)
        return ()

    lax.fori_loop(0, MAXLEN // NBUF, group_body, ())

    for b in range(NBUF):
        store_wait(b, MAXLEN - NBUF + b)


def kernel(x, input_table, fix_embedding):
    out4 = _embed_sc(x.T, input_table, fix_embedding)
    out5 = out4.reshape(MAXLEN, EBLK, NW, 8, BB)
    y = jnp.transpose(out5, (2, 4, 0, 1, 3)).reshape(BATCH, MAXLEN, EMBED)
    return y


# speedup vs baseline: 1.9294x; 1.7652x over previous
import functools

import jax
import jax.numpy as jnp
from jax import lax
from jax.experimental import pallas as pl
from jax.experimental.pallas import tpu as pltpu
from jax.experimental.pallas import tpu_sc as plsc

VOCAB = 1000000
EMBED = 64
MAXLEN = 200
BATCH = 4096
NW = 32
BB = BATCH // NW
NBUF = 4
NLANES = 16
EBLK = EMBED // 8

_mesh = plsc.VectorSubcoreMesh(core_axis_name="c", subcore_axis_name="s")


@functools.partial(
    pl.kernel,
    out_type=jax.ShapeDtypeStruct((MAXLEN, EBLK, NW, 8, BB), jnp.float32),
    mesh=_mesh,
    scratch_types=[
        pltpu.VMEM((MAXLEN, EMBED), jnp.float32),
        [pltpu.VMEM((BB,), jnp.int32) for _ in range(NBUF)],
        [pltpu.VMEM((BB, EMBED), jnp.float32) for _ in range(NBUF)],
        [pltpu.VMEM((EBLK, 1, 8, BB + 1), jnp.float32) for _ in range(NBUF)],
        [pltpu.SemaphoreType.DMA for _ in range(NBUF)],
        [pltpu.SemaphoreType.DMA for _ in range(NBUF)],
        [pltpu.SemaphoreType.DMA for _ in range(NBUF)],
    ],
    compiler_params=pltpu.CompilerParams(use_tc_tiling_on_sc=False,
                                         needs_layout_passes=False),
)
def _embed_sc(xt_hbm, table_hbm, fix_hbm, out_hbm,
              fix_v, idx_v, rows_v, outb_v, isem, gsem, ssem):
    wid = lax.axis_index("s") * 2 + lax.axis_index("c")
    b0 = wid * BB
    pltpu.sync_copy(fix_hbm, fix_v)

    def idx_start(b, t):
        pltpu.async_copy(xt_hbm.at[t, pl.ds(b0, BB)], idx_v[b], isem[b])

    def idx_wait(b, t):
        pltpu.make_async_copy(xt_hbm.at[t, pl.ds(b0, BB)], idx_v[b],
                              isem[b]).wait()

    def gather_start(b):
        pltpu.async_copy(table_hbm.at[idx_v[b]], rows_v[b], gsem[b])

    def gather_wait(b):
        pltpu.make_async_copy(table_hbm.at[idx_v[b]], rows_v[b],
                              gsem[b]).wait()

    def out_slice(t):
        return out_hbm.at[t, :, pl.ds(wid, 1)]

    def ob_slice(b):
        return outb_v[b].at[:, :, :, pl.ds(0, BB)]

    def store_start(b, t):
        pltpu.async_copy(ob_slice(b), out_slice(t), ssem[b])

    def store_wait(b, t):
        pltpu.make_async_copy(ob_slice(b), out_slice(t), ssem[b]).wait()

    def add_transpose(b, t):
        rows = rows_v[b]
        ob = outb_v[b]
        lanes = lax.iota(jnp.int32, NLANES)
        zeros = jnp.zeros((NLANES,), jnp.int32)
        e_ids = [lanes + (g * NLANES) for g in range(EMBED // NLANES)]
        eb_ids = [e >> 3 for e in e_ids]
        ei_ids = [e & 7 for e in e_ids]
        frow = [fix_v[t, pl.ds(g * NLANES, NLANES)]
                for g in range(EMBED // NLANES)]

        def b_body(bi, _):
            for u in range(2):
                brow = bi * 2 + u
                bsplat = jnp.full((NLANES,), brow, jnp.int32)
                for g in range(EMBED // NLANES):
                    vals = rows[brow, pl.ds(g * NLANES, NLANES)] + frow[g]
                    plsc.store_scatter(
                        ob, [eb_ids[g], zeros, ei_ids[g], bsplat], vals)
            return ()

        lax.fori_loop(0, BB // 2, b_body, ())

    for b in range(3):
        idx_start(b, b)
    for b in range(2):
        idx_wait(b, b)
        gather_start(b)

    def group_body(g, _):
        for b in range(NBUF):
            t = g * NBUF + b
            bi = (b + 3) % NBUF

            @pl.when(t + 3 < MAXLEN)
            def _():
                idx_start(bi, t + 3)

            gather_wait(b)
            add_transpose(b, t)
            store_start(b, t)

            bg = (b + 2) % NBUF

            @pl.when(t + 2 < MAXLEN)
            def _():
                @pl.when(t >= 2)
                def _():
                    store_wait(bg, t - 2)
                idx_wait(bg, t + 2)
                gather_start(bg)
        return ()

    lax.fori_loop(0, MAXLEN // NBUF, group_body, ())

    for b in range(NBUF):
        store_wait(b, MAXLEN - NBUF + b)


def kernel(x, input_table, fix_embedding):
    out5 = _embed_sc(x.T, input_table, fix_embedding)
    y = jnp.transpose(out5, (2, 4, 0, 1, 3)).reshape(BATCH, MAXLEN, EMBED)
    return y
